# Initial kernel scaffold; baseline (speedup 1.0000x reference)
import functools

import jax
import jax.numpy as jnp
from jax import lax
from jax.experimental import pallas as pl
from jax.experimental.pallas import tpu as pltpu
from jax.experimental.pallas import tpu_sc as plsc

NC = 2
NS = 16
NW = NC * NS
CHUNK = 80


def _sc_agg(with_payload, n, e, d, h_hbm, srcs, dsts, pay_e, out_agg, out_pay,
            acc, accp, zbuf, zbufp, sidx, didx, rows_v, pay_v, sem):
    c = lax.axis_index("c")
    s = lax.axis_index("s")
    wid = s * NC + c
    epw = e // NW
    nch = epw // CHUNK
    rpt = n // NS
    zrows = zbuf.shape[0]
    ncopy = rpt // zrows

    def _zero(i, _):
        r = i // (d // 16)
        col = (i % (d // 16)) * 16
        zbuf[r, pl.ds(col, 16)] = jnp.zeros((16,), jnp.float32)
        return _
    lax.fori_loop(0, zrows * (d // 16), _zero, None)
    if with_payload:
        def _zerop(i, _):
            zbufp[i, pl.ds(0, 16)] = jnp.zeros((16,), jnp.float32)
            return _
        lax.fori_loop(0, zrows, _zerop, None)
    for k in range(ncopy):
        r0 = s * rpt + k * zrows
        pltpu.sync_copy(zbuf, acc.at[pl.ds(r0, zrows), :])
        if with_payload:
            pltpu.sync_copy(zbufp, accp.at[pl.ds(r0, zrows), :])
    plsc.subcore_barrier()

    def _chunk(j, _):
        e0 = wid * epw + j * CHUNK
        pltpu.sync_copy(srcs.at[pl.ds(e0, CHUNK)], sidx)
        pltpu.sync_copy(dsts.at[pl.ds(e0, CHUNK)], didx)
        pltpu.async_copy(h_hbm.at[sidx], rows_v, sem).wait()
        pltpu.sync_copy(rows_v, acc.at[didx], add=True)
        if with_payload:
            pltpu.sync_copy(pay_e.at[pl.ds(e0, CHUNK), :], pay_v)
            pltpu.sync_copy(pay_v, accp.at[didx], add=True)
        return _
    lax.fori_loop(0, nch, _chunk, None)
    plsc.subcore_barrier()

    r0 = s * rpt
    pltpu.sync_copy(acc.at[pl.ds(r0, rpt), :], out_agg.at[c, pl.ds(r0, rpt), :])
    if with_payload:
        pltpu.sync_copy(accp.at[pl.ds(r0, rpt), :],
                        out_pay.at[c, pl.ds(r0, rpt), :])


def _sc_pass(h, srcs, dsts, pay_e, with_payload):
    n, d = h.shape
    e = srcs.shape[0]
    mesh = plsc.VectorSubcoreMesh(core_axis_name="c", subcore_axis_name="s",
                                  num_cores=NC, num_subcores=NS)
    zrows = 125
    out_type = [jax.ShapeDtypeStruct((NC, n, d), jnp.float32),
                jax.ShapeDtypeStruct((NC, n, 16), jnp.float32)]
    scratch = [
        pltpu.VMEM_SHARED((n, d), jnp.float32),
        pltpu.VMEM_SHARED((n, 16), jnp.float32),
        pltpu.VMEM((zrows, d), jnp.float32),
        pltpu.VMEM((zrows, 16), jnp.float32),
        pltpu.VMEM((CHUNK,), jnp.int32),
        pltpu.VMEM((CHUNK,), jnp.int32),
        pltpu.VMEM((CHUNK, d), jnp.float32),
        pltpu.VMEM((CHUNK, 16), jnp.float32),
        pltpu.SemaphoreType.DMA,
    ]
    body = functools.partial(_sc_agg, with_payload, n, e, d)
    fn = pl.kernel(body, out_type=out_type, mesh=mesh, scratch_types=scratch,
                   name="sc_agg_pay" if with_payload else "sc_agg")
    agg, pay = fn(h, srcs, dsts, pay_e)
    return agg, pay


def _tc_layer_body(h_ref, agg_ref, pay_ref, wt_ref, wb_ref, b_ref, we_ref,
                   g_ref, beta_ref, o_ref):
    h = h_ref[...]
    pay = pay_ref[0] + pay_ref[1]
    cnt = pay[:, 4:5]
    inv = jnp.where(cnt > 0, 1.0 / jnp.maximum(cnt, 1.0), 0.0)
    agg = (agg_ref[0] + agg_ref[1]) * inv
    add = jnp.dot(pay[:, :8] * inv, we_ref[...],
                  preferred_element_type=jnp.float32)
    y = (jnp.dot(h, wt_ref[...], preferred_element_type=jnp.float32)
         + jnp.dot(agg, wb_ref[...], preferred_element_type=jnp.float32)
         + b_ref[...] + add)
    m = jnp.mean(y, axis=-1, keepdims=True)
    v = jnp.mean((y - m) * (y - m), axis=-1, keepdims=True)
    o_ref[...] = (y - m) * lax.rsqrt(v + 1e-5) * g_ref[...] + beta_ref[...]


def _tc_layer(h, agg, pay, wt, wb, b, we8, gamma, beta):
    n, d = h.shape
    blk = 2000
    grid = n // blk
    fixed = lambda i: (0, 0)
    out = pl.pallas_call(
        _tc_layer_body,
        grid=(grid,),
        in_specs=[
            pl.BlockSpec((blk, d), lambda i: (i, 0)),
            pl.BlockSpec((NC, blk, d), lambda i: (0, i, 0)),
            pl.BlockSpec((NC, blk, 16), lambda i: (0, i, 0)),
            pl.BlockSpec((d, d), fixed),
            pl.BlockSpec((d, d), fixed),
            pl.BlockSpec((1, d), fixed),
            pl.BlockSpec((8, d), fixed),
            pl.BlockSpec((1, d), fixed),
            pl.BlockSpec((1, d), fixed),
        ],
        out_specs=pl.BlockSpec((blk, d), lambda i: (i, 0)),
        out_shape=jax.ShapeDtypeStruct((n, d), jnp.float32),
    )(h, agg, pay, wt, wb, b, we8, gamma, beta)
    return out


def kernel(x, edge_index, edge_attr, W1, b1, W2, b2, We, be, gamma, beta):
    n, d = x.shape
    e = edge_index.shape[1]
    de = edge_attr.shape[1]
    src = edge_index[0]
    dst = edge_index[1]
    pay_e = jnp.concatenate(
        [edge_attr, jnp.ones((e, 1), jnp.float32),
         jnp.zeros((e, 16 - de - 1), jnp.float32)], axis=-1)
    we8 = jnp.concatenate(
        [We, be[None, :], jnp.zeros((8 - de - 1, d), jnp.float32)], axis=0)

    agg1, pay = _sc_pass(x, src, dst, pay_e, True)
    h1 = _tc_layer(x, agg1, pay, W1[:d], W1[d:], b1[None, :], we8,
                   gamma[None, :], beta[None, :])
    agg2, _ = _sc_pass(h1, src, dst, pay_e, False)
    h2 = _tc_layer(h1, agg2, pay, W2[:d], W2[d:], b2[None, :], we8,
                   gamma[None, :], beta[None, :])
    return h2


# speedup vs baseline: 4.6315x; 4.6315x over previous
import functools

import jax
import jax.numpy as jnp
from jax import lax
from jax.experimental import pallas as pl
from jax.experimental.pallas import tpu as pltpu
from jax.experimental.pallas import tpu_sc as plsc

NC = 2
NS = 16
NW = NC * NS
CHUNK = 80
CHUNKP = 128
NPAY = 5


def _mesh():
    return plsc.VectorSubcoreMesh(core_axis_name="c", subcore_axis_name="s",
                                  num_cores=NC, num_subcores=NS)


def _sc_agg(np_, e, d, h_hbm, srcs, dsts, out_agg, acc, sidx, didx, rows_v,
            sem):
    c = lax.axis_index("c")
    s = lax.axis_index("s")
    wid = s * NC + c
    epw = e // NW
    nch = epw // CHUNK
    rpt = np_ // NS
    nstripe = rpt // CHUNK

    def _zero(i, _):
        rows_v[i // 8, pl.ds((i % 8) * 16, 16)] = jnp.zeros((16,), jnp.float32)
        return _
    lax.fori_loop(0, CHUNK * (d // 16), _zero, None)
    r0 = s * rpt
    for k in range(nstripe):
        pltpu.sync_copy(rows_v, acc.at[pl.ds(r0 + k * CHUNK, CHUNK), :])
    plsc.subcore_barrier()

    def _chunk(j, _):
        e0 = wid * epw + j * CHUNK
        pltpu.sync_copy(srcs.at[pl.ds(e0, CHUNK)], sidx)
        pltpu.sync_copy(dsts.at[pl.ds(e0, CHUNK)], didx)
        pltpu.async_copy(h_hbm.at[sidx], rows_v, sem).wait()
        pltpu.sync_copy(rows_v, acc.at[didx], add=True)
        return _
    lax.fori_loop(0, nch, _chunk, None)
    plsc.subcore_barrier()

    for k in range(nstripe):
        rk = r0 + k * CHUNK
        pltpu.sync_copy(acc.at[pl.ds(rk, CHUNK), :], rows_v)
        pltpu.sync_copy(rows_v, out_agg.at[c, pl.ds(rk, CHUNK), :])


def _sc_agg_pass(h, srcs, dsts):
    np_, d = h.shape
    e = srcs.shape[0]
    fn = pl.kernel(
        functools.partial(_sc_agg, np_, e, d),
        out_type=jax.ShapeDtypeStruct((NC, np_, d), jnp.float32),
        mesh=_mesh(),
        scratch_types=[
            pltpu.VMEM_SHARED((np_, d), jnp.float32),
            pltpu.VMEM((CHUNK,), jnp.int32),
            pltpu.VMEM((CHUNK,), jnp.int32),
            pltpu.VMEM((CHUNK, d), jnp.float32),
            pltpu.SemaphoreType.DMA,
        ],
        name="sc_agg")
    return fn(h, srcs, dsts)


def _sc_pay(np_, e, dsts, a0, a1, a2, a3, out_pay, acc5, didx, av0, av1, av2,
            av3):
    c = lax.axis_index("c")
    s = lax.axis_index("s")
    wid = s * NC + c
    ncht = e // CHUNKP
    nch = ncht // NW + jnp.where(wid < ncht % NW, 1, 0)

    def _zero(i, _):
        acc5[pl.ds(i * 16, 16)] = jnp.zeros((16,), jnp.float32)
        return _
    lax.fori_loop(0, (NPAY * np_) // 16, _zero, None)

    ones = jnp.ones((16,), jnp.float32)

    def _chunk(t, _):
        e0 = (wid + NW * t) * CHUNKP
        pltpu.sync_copy(dsts.at[pl.ds(e0, CHUNKP)], didx)
        pltpu.sync_copy(a0.at[pl.ds(e0, CHUNKP)], av0)
        pltpu.sync_copy(a1.at[pl.ds(e0, CHUNKP)], av1)
        pltpu.sync_copy(a2.at[pl.ds(e0, CHUNKP)], av2)
        pltpu.sync_copy(a3.at[pl.ds(e0, CHUNKP)], av3)
        def _grp(g, __):
            sl = pl.ds(g * 16, 16)
            base = didx[sl] * NPAY
            plsc.addupdate_scatter(acc5, [base], av0[sl])
            plsc.addupdate_scatter(acc5, [base + 1], av1[sl])
            plsc.addupdate_scatter(acc5, [base + 2], av2[sl])
            plsc.addupdate_scatter(acc5, [base + 3], av3[sl])
            plsc.addupdate_scatter(acc5, [base + 4], ones)
            return __
        lax.fori_loop(0, CHUNKP // 16, _grp, None)
        return _
    lax.fori_loop(0, nch, _chunk, None)

    pltpu.sync_copy(acc5, out_pay.at[c, s])


def _sc_pay_pass(np_, dsts, edge_attr):
    e = dsts.shape[0]
    fn = pl.kernel(
        functools.partial(_sc_pay, np_, e),
        out_type=jax.ShapeDtypeStruct((NC, NS, NPAY * np_), jnp.float32),
        mesh=_mesh(),
        scratch_types=[
            pltpu.VMEM((NPAY * np_,), jnp.float32),
            pltpu.VMEM((CHUNKP,), jnp.int32),
            pltpu.VMEM((CHUNKP,), jnp.float32),
            pltpu.VMEM((CHUNKP,), jnp.float32),
            pltpu.VMEM((CHUNKP,), jnp.float32),
            pltpu.VMEM((CHUNKP,), jnp.float32),
        ],
        compiler_params=pltpu.CompilerParams(needs_layout_passes=False),
        name="sc_pay")
    return fn(dsts, edge_attr[:, 0], edge_attr[:, 1], edge_attr[:, 2],
              edge_attr[:, 3])


def _tc_payred_body(p_ref, o_ref):
    acc = p_ref[0, 0]
    for i in range(NC):
        for j in range(NS):
            if i or j:
                acc = acc + p_ref[i, j]
    o_ref[...] = acc


def _tc_payred(pay):
    nc, ns, rows, d = pay.shape
    blk = 80
    return pl.pallas_call(
        _tc_payred_body,
        grid=(rows // blk,),
        in_specs=[pl.BlockSpec((NC, NS, blk, d), lambda i: (0, 0, i, 0))],
        out_specs=pl.BlockSpec((blk, d), lambda i: (i, 0)),
        out_shape=jax.ShapeDtypeStruct((rows, d), jnp.float32),
    )(pay)


def _tc_layer_body(h_ref, agg_ref, pay_ref, wt_ref, wb_ref, b_ref, we_ref,
                   be_ref, g_ref, beta_ref, o_ref):
    h = h_ref[...]
    ps = pay_ref[...]
    cnt = ps[:, 4:5]
    inv = jnp.where(cnt > 0, 1.0 / jnp.maximum(cnt, 1.0), 0.0)
    agg = (agg_ref[0] + agg_ref[1]) * inv
    attr_mean = ps[:, :4] * inv
    add = jnp.where(
        cnt > 0,
        jnp.dot(attr_mean, we_ref[:4], preferred_element_type=jnp.float32)
        + be_ref[...],
        0.0)
    y = (jnp.dot(h, wt_ref[...], preferred_element_type=jnp.float32)
         + jnp.dot(agg, wb_ref[...], preferred_element_type=jnp.float32)
         + b_ref[...] + add)
    m = jnp.mean(y, axis=-1, keepdims=True)
    v = jnp.mean((y - m) * (y - m), axis=-1, keepdims=True)
    o_ref[...] = (y - m) * lax.rsqrt(v + 1e-5) * g_ref[...] + beta_ref[...]


def _tc_layer(h, agg, pay4, wt, wb, b, we8, be, gamma, beta):
    np_, d = h.shape
    blk = 2048
    grid = np_ // blk
    fixed = lambda i: (0, 0)
    out = pl.pallas_call(
        _tc_layer_body,
        grid=(grid,),
        in_specs=[
            pl.BlockSpec((blk, d), lambda i: (i, 0)),
            pl.BlockSpec((NC, blk, d), lambda i: (0, i, 0)),
            pl.BlockSpec((blk, NPAY), lambda i: (i, 0)),
            pl.BlockSpec((d, d), fixed),
            pl.BlockSpec((d, d), fixed),
            pl.BlockSpec((1, d), fixed),
            pl.BlockSpec((8, d), fixed),
            pl.BlockSpec((1, d), fixed),
            pl.BlockSpec((1, d), fixed),
            pl.BlockSpec((1, d), fixed),
        ],
        out_specs=pl.BlockSpec((blk, d), lambda i: (i, 0)),
        out_shape=jax.ShapeDtypeStruct((np_, d), jnp.float32),
    )(h, agg, pay4, wt, wb, b, we8, be, gamma, beta)
    return out


def kernel(x, edge_index, edge_attr, W1, b1, W2, b2, We, be, gamma, beta):
    n, d = x.shape
    e = edge_index.shape[1]
    de = edge_attr.shape[1]
    src = edge_index[0]
    dst = edge_index[1]
    np_ = ((n + NS * CHUNK - 1) // (NS * CHUNK)) * (NS * CHUNK)
    xp = jnp.concatenate([x, jnp.zeros((np_ - n, d), jnp.float32)], axis=0)
    we8 = jnp.concatenate(
        [We, jnp.zeros((8 - de, d), jnp.float32)], axis=0)

    pay = _sc_pay_pass(np_, dst, edge_attr)
    rows = (NPAY * np_) // 128
    pay4 = _tc_payred(pay.reshape(NC, NS, rows, 128)).reshape(np_, NPAY)
    agg1 = _sc_agg_pass(xp, src, dst)
    h1 = _tc_layer(xp, agg1, pay4, W1[:d], W1[d:], b1[None, :], we8,
                   be[None, :], gamma[None, :], beta[None, :])
    agg2 = _sc_agg_pass(h1, src, dst)
    h2 = _tc_layer(h1, agg2, pay4, W2[:d], W2[d:], b2[None, :], we8,
                   be[None, :], gamma[None, :], beta[None, :])
    return h2[:n]


# speedup vs baseline: 7.4105x; 1.6000x over previous
import functools

import jax
import jax.numpy as jnp
from jax import lax
from jax.experimental import pallas as pl
from jax.experimental.pallas import tpu as pltpu
from jax.experimental.pallas import tpu_sc as plsc

NC = 2
NS = 16
NW = NC * NS
CHUNK = 80
CHUNKP = 512
NPAY = 5


def _mesh():
    return plsc.VectorSubcoreMesh(core_axis_name="c", subcore_axis_name="s",
                                  num_cores=NC, num_subcores=NS)


def _sc_agg(np_, e, d, h_hbm, srcs, dsts, out_agg, acc, sidx0, didx0, sidx1,
            didx1, rows_v0, rows_v1, sem0, sem1):
    c = lax.axis_index("c")
    s = lax.axis_index("s")
    wid = s * NC + c
    epw = e // NW
    nch = epw // CHUNK
    rpt = np_ // NS
    nstripe = rpt // CHUNK

    def _zero(i, _):
        rows_v0[i // 8, pl.ds((i % 8) * 16, 16)] = jnp.zeros((16,),
                                                             jnp.float32)
        return _
    lax.fori_loop(0, CHUNK * (d // 16), _zero, None)
    r0 = s * rpt
    for k in range(nstripe):
        pltpu.sync_copy(rows_v0, acc.at[pl.ds(r0 + k * CHUNK, CHUNK), :])
    plsc.subcore_barrier()

    ebase = wid * epw

    def _pair(p, _):
        e0 = ebase + (2 * p) * CHUNK
        e1 = e0 + CHUNK
        pltpu.sync_copy(srcs.at[pl.ds(e0, CHUNK)], sidx0)
        pltpu.sync_copy(dsts.at[pl.ds(e0, CHUNK)], didx0)
        g0 = pltpu.async_copy(h_hbm.at[sidx0], rows_v0, sem0)
        pltpu.sync_copy(srcs.at[pl.ds(e1, CHUNK)], sidx1)
        pltpu.sync_copy(dsts.at[pl.ds(e1, CHUNK)], didx1)
        g1 = pltpu.async_copy(h_hbm.at[sidx1], rows_v1, sem1)
        g0.wait()
        pltpu.sync_copy(rows_v0, acc.at[didx0], add=True)
        g1.wait()
        pltpu.sync_copy(rows_v1, acc.at[didx1], add=True)
        return _
    lax.fori_loop(0, nch // 2, _pair, None)
    if nch % 2:
        e0 = ebase + (nch - 1) * CHUNK
        pltpu.sync_copy(srcs.at[pl.ds(e0, CHUNK)], sidx0)
        pltpu.sync_copy(dsts.at[pl.ds(e0, CHUNK)], didx0)
        pltpu.async_copy(h_hbm.at[sidx0], rows_v0, sem0).wait()
        pltpu.sync_copy(rows_v0, acc.at[didx0], add=True)
    plsc.subcore_barrier()

    for k in range(nstripe):
        rk = r0 + k * CHUNK
        pltpu.sync_copy(acc.at[pl.ds(rk, CHUNK), :], rows_v0)
        pltpu.sync_copy(rows_v0, out_agg.at[c, pl.ds(rk, CHUNK), :])


def _sc_agg_pass(h, srcs, dsts):
    np_, d = h.shape
    e = srcs.shape[0]
    fn = pl.kernel(
        functools.partial(_sc_agg, np_, e, d),
        out_type=jax.ShapeDtypeStruct((NC, np_, d), jnp.float32),
        mesh=_mesh(),
        scratch_types=[
            pltpu.VMEM_SHARED((np_, d), jnp.float32),
            pltpu.VMEM((CHUNK,), jnp.int32),
            pltpu.VMEM((CHUNK,), jnp.int32),
            pltpu.VMEM((CHUNK,), jnp.int32),
            pltpu.VMEM((CHUNK,), jnp.int32),
            pltpu.VMEM((CHUNK, d), jnp.float32),
            pltpu.VMEM((CHUNK, d), jnp.float32),
            pltpu.SemaphoreType.DMA,
            pltpu.SemaphoreType.DMA,
        ],
        name="sc_agg")
    return fn(h, srcs, dsts)


def _sc_pay(np_, e, dsts, a0, a1, a2, a3, out_pay, acc5, didx, av0, av1, av2,
            av3, semp):
    c = lax.axis_index("c")
    s = lax.axis_index("s")
    wid = s * NC + c
    ncht = e // CHUNKP
    nch = ncht // NW + jnp.where(wid < ncht % NW, 1, 0)

    def _zero(i, _):
        acc5[pl.ds(i * 16, 16)] = jnp.zeros((16,), jnp.float32)
        return _
    lax.fori_loop(0, (NPAY * np_) // 16, _zero, None)

    ones = jnp.ones((16,), jnp.float32)

    def _chunk(t, _):
        e0 = (wid + NW * t) * CHUNKP
        cps = [pltpu.async_copy(dsts.at[pl.ds(e0, CHUNKP)], didx, semp),
               pltpu.async_copy(a0.at[pl.ds(e0, CHUNKP)], av0, semp),
               pltpu.async_copy(a1.at[pl.ds(e0, CHUNKP)], av1, semp),
               pltpu.async_copy(a2.at[pl.ds(e0, CHUNKP)], av2, semp),
               pltpu.async_copy(a3.at[pl.ds(e0, CHUNKP)], av3, semp)]
        for cp in cps:
            cp.wait()
        def _grp(g, __):
            sl = pl.ds(g * 16, 16)
            base = didx[sl] * NPAY
            plsc.addupdate_scatter(acc5, [base], av0[sl])
            plsc.addupdate_scatter(acc5, [base + 1], av1[sl])
            plsc.addupdate_scatter(acc5, [base + 2], av2[sl])
            plsc.addupdate_scatter(acc5, [base + 3], av3[sl])
            plsc.addupdate_scatter(acc5, [base + 4], ones)
            return __
        lax.fori_loop(0, CHUNKP // 16, _grp, None)
        return _
    lax.fori_loop(0, nch, _chunk, None)

    pltpu.sync_copy(acc5, out_pay.at[c, s])


def _sc_pay_pass(np_, dsts, edge_attr):
    e = dsts.shape[0]
    fn = pl.kernel(
        functools.partial(_sc_pay, np_, e),
        out_type=jax.ShapeDtypeStruct((NC, NS, NPAY * np_), jnp.float32),
        mesh=_mesh(),
        scratch_types=[
            pltpu.VMEM((NPAY * np_,), jnp.float32),
            pltpu.VMEM((CHUNKP,), jnp.int32),
            pltpu.VMEM((CHUNKP,), jnp.float32),
            pltpu.VMEM((CHUNKP,), jnp.float32),
            pltpu.VMEM((CHUNKP,), jnp.float32),
            pltpu.VMEM((CHUNKP,), jnp.float32),
            pltpu.SemaphoreType.DMA,
        ],
        compiler_params=pltpu.CompilerParams(needs_layout_passes=False),
        name="sc_pay")
    return fn(dsts, edge_attr[:, 0], edge_attr[:, 1], edge_attr[:, 2],
              edge_attr[:, 3])


def _tc_payred_body(p_ref, o_ref):
    acc = p_ref[0, 0]
    for i in range(NC):
        for j in range(NS):
            if i or j:
                acc = acc + p_ref[i, j]
    o_ref[...] = acc


def _tc_payred(pay):
    nc, ns, rows, d = pay.shape
    blk = 80
    return pl.pallas_call(
        _tc_payred_body,
        grid=(rows // blk,),
        in_specs=[pl.BlockSpec((NC, NS, blk, d), lambda i: (0, 0, i, 0))],
        out_specs=pl.BlockSpec((blk, d), lambda i: (i, 0)),
        out_shape=jax.ShapeDtypeStruct((rows, d), jnp.float32),
    )(pay)


def _tc_layer_body(h_ref, agg_ref, pay_ref, wt_ref, wb_ref, b_ref, we_ref,
                   be_ref, g_ref, beta_ref, o_ref):
    h = h_ref[...]
    ps = pay_ref[...]
    cnt = ps[:, 4:5]
    inv = jnp.where(cnt > 0, 1.0 / jnp.maximum(cnt, 1.0), 0.0)
    agg = (agg_ref[0] + agg_ref[1]) * inv
    attr_mean = ps[:, :4] * inv
    add = jnp.where(
        cnt > 0,
        jnp.dot(attr_mean, we_ref[:4], preferred_element_type=jnp.float32)
        + be_ref[...],
        0.0)
    y = (jnp.dot(h, wt_ref[...], preferred_element_type=jnp.float32)
         + jnp.dot(agg, wb_ref[...], preferred_element_type=jnp.float32)
         + b_ref[...] + add)
    m = jnp.mean(y, axis=-1, keepdims=True)
    v = jnp.mean((y - m) * (y - m), axis=-1, keepdims=True)
    o_ref[...] = (y - m) * lax.rsqrt(v + 1e-5) * g_ref[...] + beta_ref[...]


def _tc_layer(h, agg, pay4, wt, wb, b, we8, be, gamma, beta):
    np_, d = h.shape
    blk = 2048
    grid = np_ // blk
    fixed = lambda i: (0, 0)
    out = pl.pallas_call(
        _tc_layer_body,
        grid=(grid,),
        in_specs=[
            pl.BlockSpec((blk, d), lambda i: (i, 0)),
            pl.BlockSpec((NC, blk, d), lambda i: (0, i, 0)),
            pl.BlockSpec((blk, NPAY), lambda i: (i, 0)),
            pl.BlockSpec((d, d), fixed),
            pl.BlockSpec((d, d), fixed),
            pl.BlockSpec((1, d), fixed),
            pl.BlockSpec((8, d), fixed),
            pl.BlockSpec((1, d), fixed),
            pl.BlockSpec((1, d), fixed),
            pl.BlockSpec((1, d), fixed),
        ],
        out_specs=pl.BlockSpec((blk, d), lambda i: (i, 0)),
        out_shape=jax.ShapeDtypeStruct((np_, d), jnp.float32),
    )(h, agg, pay4, wt, wb, b, we8, be, gamma, beta)
    return out


def kernel(x, edge_index, edge_attr, W1, b1, W2, b2, We, be, gamma, beta):
    n, d = x.shape
    e = edge_index.shape[1]
    de = edge_attr.shape[1]
    src = edge_index[0]
    dst = edge_index[1]
    np_ = ((n + NS * CHUNK - 1) // (NS * CHUNK)) * (NS * CHUNK)
    xp = jnp.concatenate([x, jnp.zeros((np_ - n, d), jnp.float32)], axis=0)
    we8 = jnp.concatenate(
        [We, jnp.zeros((8 - de, d), jnp.float32)], axis=0)

    pay = _sc_pay_pass(np_, dst, edge_attr)
    rows = (NPAY * np_) // 128
    pay4 = _tc_payred(pay.reshape(NC, NS, rows, 128)).reshape(np_, NPAY)
    agg1 = _sc_agg_pass(xp, src, dst)
    h1 = _tc_layer(xp, agg1, pay4, W1[:d], W1[d:], b1[None, :], we8,
                   be[None, :], gamma[None, :], beta[None, :])
    agg2 = _sc_agg_pass(h1, src, dst)
    h2 = _tc_layer(h1, agg2, pay4, W2[:d], W2[d:], b2[None, :], we8,
                   be[None, :], gamma[None, :], beta[None, :])
    return h2[:n]


# speedup vs baseline: 8.5603x; 1.1552x over previous
import functools

import jax
import jax.numpy as jnp
from jax import lax
from jax.experimental import pallas as pl
from jax.experimental.pallas import tpu as pltpu
from jax.experimental.pallas import tpu_sc as plsc

NC = 2
NS = 16
NW = NC * NS
CHUNK = 80
CHUNKP = 512
NPAY = 5


def _mesh():
    return plsc.VectorSubcoreMesh(core_axis_name="c", subcore_axis_name="s",
                                  num_cores=NC, num_subcores=NS)


NBUF = 4


def _sc_agg(np_, e, d, h_hbm, srcs, dsts, out_agg, acc, *bufs):
    sidx = bufs[0:NBUF]
    didx = bufs[NBUF:2 * NBUF]
    rows = bufs[2 * NBUF:3 * NBUF]
    sems = bufs[3 * NBUF:4 * NBUF]
    c = lax.axis_index("c")
    s = lax.axis_index("s")
    wid = s * NC + c
    epw = e // NW
    nch = epw // CHUNK
    rpt = np_ // NS
    nstripe = rpt // CHUNK

    def _zero(i, _):
        rows[0][i // 8, pl.ds((i % 8) * 16, 16)] = jnp.zeros((16,),
                                                             jnp.float32)
        return _
    lax.fori_loop(0, CHUNK * (d // 16), _zero, None)
    r0 = s * rpt
    for k in range(nstripe):
        pltpu.sync_copy(rows[0], acc.at[pl.ds(r0 + k * CHUNK, CHUNK), :])
    plsc.subcore_barrier()

    ebase = wid * epw
    for b in range(min(NBUF, nch)):
        e0 = ebase + b * CHUNK
        pltpu.sync_copy(srcs.at[pl.ds(e0, CHUNK)], sidx[b])
        pltpu.sync_copy(dsts.at[pl.ds(e0, CHUNK)], didx[b])
        pltpu.async_copy(h_hbm.at[sidx[b]], rows[b], sems[b])
    nq = nch // NBUF

    def _quad(q, _):
        for b in range(NBUF):
            pltpu.make_async_copy(h_hbm.at[sidx[b]], rows[b], sems[b]).wait()
            pltpu.sync_copy(rows[b], acc.at[didx[b]], add=True)
            nxt = q * NBUF + b + NBUF
            @pl.when(nxt < nch)
            def _(b=b, nxt=nxt):
                e0 = ebase + nxt * CHUNK
                pltpu.sync_copy(srcs.at[pl.ds(e0, CHUNK)], sidx[b])
                pltpu.sync_copy(dsts.at[pl.ds(e0, CHUNK)], didx[b])
                pltpu.async_copy(h_hbm.at[sidx[b]], rows[b], sems[b])
        return _
    lax.fori_loop(0, nq, _quad, None)
    for ch in range(nq * NBUF, nch):
        b = ch % NBUF
        pltpu.make_async_copy(h_hbm.at[sidx[b]], rows[b], sems[b]).wait()
        pltpu.sync_copy(rows[b], acc.at[didx[b]], add=True)
    plsc.subcore_barrier()

    for k in range(nstripe):
        rk = r0 + k * CHUNK
        pltpu.sync_copy(acc.at[pl.ds(rk, CHUNK), :], rows[0])
        pltpu.sync_copy(rows[0], out_agg.at[c, pl.ds(rk, CHUNK), :])


def _sc_agg_pass(h, srcs, dsts):
    np_, d = h.shape
    e = srcs.shape[0]
    scratch = [pltpu.VMEM_SHARED((np_, d), jnp.float32)]
    scratch += [pltpu.VMEM((CHUNK,), jnp.int32) for _ in range(NBUF)]
    scratch += [pltpu.VMEM((CHUNK,), jnp.int32) for _ in range(NBUF)]
    scratch += [pltpu.VMEM((CHUNK, d), jnp.float32) for _ in range(NBUF)]
    scratch += [pltpu.SemaphoreType.DMA for _ in range(NBUF)]
    fn = pl.kernel(
        functools.partial(_sc_agg, np_, e, d),
        out_type=jax.ShapeDtypeStruct((NC, np_, d), jnp.float32),
        mesh=_mesh(),
        scratch_types=scratch,
        name="sc_agg")
    return fn(h, srcs, dsts)


def _sc_pay(np_, e, dsts, a0, a1, a2, a3, out_pay, acc5, didx, av0, av1, av2,
            av3, semp):
    c = lax.axis_index("c")
    s = lax.axis_index("s")
    wid = s * NC + c
    ncht = e // CHUNKP
    nch = ncht // NW + jnp.where(wid < ncht % NW, 1, 0)

    def _zero(i, _):
        acc5[pl.ds(i * 16, 16)] = jnp.zeros((16,), jnp.float32)
        return _
    lax.fori_loop(0, (NPAY * np_) // 16, _zero, None)

    ones = jnp.ones((16,), jnp.float32)

    def _chunk(t, _):
        e0 = (wid + NW * t) * CHUNKP
        cps = [pltpu.async_copy(dsts.at[pl.ds(e0, CHUNKP)], didx, semp),
               pltpu.async_copy(a0.at[pl.ds(e0, CHUNKP)], av0, semp),
               pltpu.async_copy(a1.at[pl.ds(e0, CHUNKP)], av1, semp),
               pltpu.async_copy(a2.at[pl.ds(e0, CHUNKP)], av2, semp),
               pltpu.async_copy(a3.at[pl.ds(e0, CHUNKP)], av3, semp)]
        for cp in cps:
            cp.wait()
        def _grp(g, __):
            sl = pl.ds(g * 16, 16)
            base = didx[sl] * NPAY
            plsc.addupdate_scatter(acc5, [base], av0[sl])
            plsc.addupdate_scatter(acc5, [base + 1], av1[sl])
            plsc.addupdate_scatter(acc5, [base + 2], av2[sl])
            plsc.addupdate_scatter(acc5, [base + 3], av3[sl])
            plsc.addupdate_scatter(acc5, [base + 4], ones)
            return __
        lax.fori_loop(0, CHUNKP // 16, _grp, None)
        return _
    lax.fori_loop(0, nch, _chunk, None)

    pltpu.sync_copy(acc5, out_pay.at[c, s])


def _sc_pay_pass(np_, dsts, edge_attr):
    e = dsts.shape[0]
    fn = pl.kernel(
        functools.partial(_sc_pay, np_, e),
        out_type=jax.ShapeDtypeStruct((NC, NS, NPAY * np_), jnp.float32),
        mesh=_mesh(),
        scratch_types=[
            pltpu.VMEM((NPAY * np_,), jnp.float32),
            pltpu.VMEM((CHUNKP,), jnp.int32),
            pltpu.VMEM((CHUNKP,), jnp.float32),
            pltpu.VMEM((CHUNKP,), jnp.float32),
            pltpu.VMEM((CHUNKP,), jnp.float32),
            pltpu.VMEM((CHUNKP,), jnp.float32),
            pltpu.SemaphoreType.DMA,
        ],
        compiler_params=pltpu.CompilerParams(needs_layout_passes=False),
        name="sc_pay")
    return fn(dsts, edge_attr[:, 0], edge_attr[:, 1], edge_attr[:, 2],
              edge_attr[:, 3])


def _tc_payred_body(p_ref, o_ref):
    acc = p_ref[0, 0]
    for i in range(NC):
        for j in range(NS):
            if i or j:
                acc = acc + p_ref[i, j]
    o_ref[...] = acc


def _tc_payred(pay):
    nc, ns, rows, d = pay.shape
    blk = 80
    return pl.pallas_call(
        _tc_payred_body,
        grid=(rows // blk,),
        in_specs=[pl.BlockSpec((NC, NS, blk, d), lambda i: (0, 0, i, 0))],
        out_specs=pl.BlockSpec((blk, d), lambda i: (i, 0)),
        out_shape=jax.ShapeDtypeStruct((rows, d), jnp.float32),
    )(pay)


def _tc_layer_body(h_ref, agg_ref, pay_ref, wt_ref, wb_ref, b_ref, we_ref,
                   be_ref, g_ref, beta_ref, o_ref):
    h = h_ref[...]
    ps = pay_ref[...]
    cnt = ps[:, 4:5]
    inv = jnp.where(cnt > 0, 1.0 / jnp.maximum(cnt, 1.0), 0.0)
    agg = (agg_ref[0] + agg_ref[1]) * inv
    attr_mean = ps[:, :4] * inv
    add = jnp.where(
        cnt > 0,
        jnp.dot(attr_mean, we_ref[:4], preferred_element_type=jnp.float32)
        + be_ref[...],
        0.0)
    y = (jnp.dot(h, wt_ref[...], preferred_element_type=jnp.float32)
         + jnp.dot(agg, wb_ref[...], preferred_element_type=jnp.float32)
         + b_ref[...] + add)
    m = jnp.mean(y, axis=-1, keepdims=True)
    v = jnp.mean((y - m) * (y - m), axis=-1, keepdims=True)
    o_ref[...] = (y - m) * lax.rsqrt(v + 1e-5) * g_ref[...] + beta_ref[...]


def _tc_layer(h, agg, pay4, wt, wb, b, we8, be, gamma, beta):
    np_, d = h.shape
    blk = 2048
    grid = np_ // blk
    fixed = lambda i: (0, 0)
    out = pl.pallas_call(
        _tc_layer_body,
        grid=(grid,),
        in_specs=[
            pl.BlockSpec((blk, d), lambda i: (i, 0)),
            pl.BlockSpec((NC, blk, d), lambda i: (0, i, 0)),
            pl.BlockSpec((blk, NPAY), lambda i: (i, 0)),
            pl.BlockSpec((d, d), fixed),
            pl.BlockSpec((d, d), fixed),
            pl.BlockSpec((1, d), fixed),
            pl.BlockSpec((8, d), fixed),
            pl.BlockSpec((1, d), fixed),
            pl.BlockSpec((1, d), fixed),
            pl.BlockSpec((1, d), fixed),
        ],
        out_specs=pl.BlockSpec((blk, d), lambda i: (i, 0)),
        out_shape=jax.ShapeDtypeStruct((np_, d), jnp.float32),
    )(h, agg, pay4, wt, wb, b, we8, be, gamma, beta)
    return out


def kernel(x, edge_index, edge_attr, W1, b1, W2, b2, We, be, gamma, beta):
    n, d = x.shape
    e = edge_index.shape[1]
    de = edge_attr.shape[1]
    src = edge_index[0]
    dst = edge_index[1]
    np_ = ((n + NS * CHUNK - 1) // (NS * CHUNK)) * (NS * CHUNK)
    xp = jnp.concatenate([x, jnp.zeros((np_ - n, d), jnp.float32)], axis=0)
    we8 = jnp.concatenate(
        [We, jnp.zeros((8 - de, d), jnp.float32)], axis=0)

    pay = _sc_pay_pass(np_, dst, edge_attr)
    rows = (NPAY * np_) // 128
    pay4 = _tc_payred(pay.reshape(NC, NS, rows, 128)).reshape(np_, NPAY)
    agg1 = _sc_agg_pass(xp, src, dst)
    h1 = _tc_layer(xp, agg1, pay4, W1[:d], W1[d:], b1[None, :], we8,
                   be[None, :], gamma[None, :], beta[None, :])
    agg2 = _sc_agg_pass(h1, src, dst)
    h2 = _tc_layer(h1, agg2, pay4, W2[:d], W2[d:], b2[None, :], we8,
                   be[None, :], gamma[None, :], beta[None, :])
    return h2[:n]


# speedup vs baseline: 8.5645x; 1.0005x over previous
import functools

import jax
import jax.numpy as jnp
from jax import lax
from jax.experimental import pallas as pl
from jax.experimental.pallas import tpu as pltpu
from jax.experimental.pallas import tpu_sc as plsc

NC = 2
NS = 16
NW = NC * NS
CHUNK = 80
CHUNKP = 512
NPAY = 5


def _mesh():
    return plsc.VectorSubcoreMesh(core_axis_name="c", subcore_axis_name="s",
                                  num_cores=NC, num_subcores=NS)


NBUF = 4


def _sc_agg(np_, e, d, h_hbm, srcs, dsts, out_agg, acc, *bufs):
    sidx = bufs[0:NBUF]
    didx = bufs[NBUF:2 * NBUF]
    rows = bufs[2 * NBUF:3 * NBUF]
    sems = bufs[3 * NBUF:4 * NBUF]
    ssems = bufs[4 * NBUF:5 * NBUF]
    c = lax.axis_index("c")
    s = lax.axis_index("s")
    wid = s * NC + c
    epw = e // NW
    nch = epw // CHUNK
    rpt = np_ // NS
    nstripe = rpt // CHUNK

    def _zero(i, _):
        rows[0][i // 8, pl.ds((i % 8) * 16, 16)] = jnp.zeros((16,),
                                                             jnp.float32)
        return _
    lax.fori_loop(0, CHUNK * (d // 16), _zero, None)
    r0 = s * rpt
    for k in range(nstripe):
        pltpu.sync_copy(rows[0], acc.at[pl.ds(r0 + k * CHUNK, CHUNK), :])
    plsc.subcore_barrier()

    ebase = wid * epw
    for b in range(min(NBUF, nch)):
        e0 = ebase + b * CHUNK
        pltpu.sync_copy(srcs.at[pl.ds(e0, CHUNK)], sidx[b])
        pltpu.sync_copy(dsts.at[pl.ds(e0, CHUNK)], didx[b])
        pltpu.async_copy(h_hbm.at[sidx[b]], rows[b], sems[b])
    nq = nch // NBUF

    def _quad(q, _):
        for b in range(NBUF):
            pltpu.make_async_copy(h_hbm.at[sidx[b]], rows[b], sems[b]).wait()
            pltpu.async_copy(rows[b], acc.at[didx[b]], ssems[b], add=True)
            nxt = q * NBUF + b + NBUF
            @pl.when(nxt < nch)
            def _(b=b, nxt=nxt):
                e0 = ebase + nxt * CHUNK
                pltpu.make_async_copy(rows[b], acc.at[didx[b]],
                                      ssems[b]).wait()
                pltpu.sync_copy(srcs.at[pl.ds(e0, CHUNK)], sidx[b])
                pltpu.sync_copy(dsts.at[pl.ds(e0, CHUNK)], didx[b])
                pltpu.async_copy(h_hbm.at[sidx[b]], rows[b], sems[b])
        return _
    lax.fori_loop(0, nq, _quad, None)
    for ch in range(nq * NBUF, nch):
        b = ch % NBUF
        pltpu.make_async_copy(h_hbm.at[sidx[b]], rows[b], sems[b]).wait()
        pltpu.async_copy(rows[b], acc.at[didx[b]], ssems[b], add=True)
    for b in range(NBUF):
        pltpu.make_async_copy(rows[b], acc.at[didx[b]], ssems[b]).wait()
    plsc.subcore_barrier()

    for k in range(nstripe):
        rk = r0 + k * CHUNK
        pltpu.sync_copy(acc.at[pl.ds(rk, CHUNK), :], rows[0])
        pltpu.sync_copy(rows[0], out_agg.at[c, pl.ds(rk, CHUNK), :])


def _sc_agg_pass(h, srcs, dsts):
    np_, d = h.shape
    e = srcs.shape[0]
    scratch = [pltpu.VMEM_SHARED((np_, d), jnp.float32)]
    scratch += [pltpu.VMEM((CHUNK,), jnp.int32) for _ in range(NBUF)]
    scratch += [pltpu.VMEM((CHUNK,), jnp.int32) for _ in range(NBUF)]
    scratch += [pltpu.VMEM((CHUNK, d), jnp.float32) for _ in range(NBUF)]
    scratch += [pltpu.SemaphoreType.DMA for _ in range(2 * NBUF)]
    fn = pl.kernel(
        functools.partial(_sc_agg, np_, e, d),
        out_type=jax.ShapeDtypeStruct((NC, np_, d), jnp.float32),
        mesh=_mesh(),
        scratch_types=scratch,
        name="sc_agg")
    return fn(h, srcs, dsts)


def _sc_pay(np_, e, dsts, a0, a1, a2, a3, out_pay, acc5, didx, av0, av1, av2,
            av3, semp):
    c = lax.axis_index("c")
    s = lax.axis_index("s")
    wid = s * NC + c
    ncht = e // CHUNKP
    nch = ncht // NW + jnp.where(wid < ncht % NW, 1, 0)

    def _zero(i, _):
        acc5[pl.ds(i * 16, 16)] = jnp.zeros((16,), jnp.float32)
        return _
    lax.fori_loop(0, (NPAY * np_) // 16, _zero, None)

    ones = jnp.ones((16,), jnp.float32)

    def _chunk(t, _):
        e0 = (wid + NW * t) * CHUNKP
        cps = [pltpu.async_copy(dsts.at[pl.ds(e0, CHUNKP)], didx, semp),
               pltpu.async_copy(a0.at[pl.ds(e0, CHUNKP)], av0, semp),
               pltpu.async_copy(a1.at[pl.ds(e0, CHUNKP)], av1, semp),
               pltpu.async_copy(a2.at[pl.ds(e0, CHUNKP)], av2, semp),
               pltpu.async_copy(a3.at[pl.ds(e0, CHUNKP)], av3, semp)]
        for cp in cps:
            cp.wait()
        def _grp(g, __):
            sl = pl.ds(g * 16, 16)
            base = didx[sl] * NPAY
            plsc.addupdate_scatter(acc5, [base], av0[sl])
            plsc.addupdate_scatter(acc5, [base + 1], av1[sl])
            plsc.addupdate_scatter(acc5, [base + 2], av2[sl])
            plsc.addupdate_scatter(acc5, [base + 3], av3[sl])
            plsc.addupdate_scatter(acc5, [base + 4], ones)
            return __
        lax.fori_loop(0, CHUNKP // 16, _grp, None)
        return _
    lax.fori_loop(0, nch, _chunk, None)

    pltpu.sync_copy(acc5, out_pay.at[c, s])


def _sc_pay_pass(np_, dsts, edge_attr):
    e = dsts.shape[0]
    fn = pl.kernel(
        functools.partial(_sc_pay, np_, e),
        out_type=jax.ShapeDtypeStruct((NC, NS, NPAY * np_), jnp.float32),
        mesh=_mesh(),
        scratch_types=[
            pltpu.VMEM((NPAY * np_,), jnp.float32),
            pltpu.VMEM((CHUNKP,), jnp.int32),
            pltpu.VMEM((CHUNKP,), jnp.float32),
            pltpu.VMEM((CHUNKP,), jnp.float32),
            pltpu.VMEM((CHUNKP,), jnp.float32),
            pltpu.VMEM((CHUNKP,), jnp.float32),
            pltpu.SemaphoreType.DMA,
        ],
        compiler_params=pltpu.CompilerParams(needs_layout_passes=False),
        name="sc_pay")
    return fn(dsts, edge_attr[:, 0], edge_attr[:, 1], edge_attr[:, 2],
              edge_attr[:, 3])


def _tc_payred_body(p_ref, o_ref):
    acc = p_ref[0, 0]
    for i in range(NC):
        for j in range(NS):
            if i or j:
                acc = acc + p_ref[i, j]
    o_ref[...] = acc


def _tc_payred(pay):
    nc, ns, rows, d = pay.shape
    blk = 80
    return pl.pallas_call(
        _tc_payred_body,
        grid=(rows // blk,),
        in_specs=[pl.BlockSpec((NC, NS, blk, d), lambda i: (0, 0, i, 0))],
        out_specs=pl.BlockSpec((blk, d), lambda i: (i, 0)),
        out_shape=jax.ShapeDtypeStruct((rows, d), jnp.float32),
    )(pay)


def _tc_layer_body(h_ref, agg_ref, pay_ref, wt_ref, wb_ref, b_ref, we_ref,
                   be_ref, g_ref, beta_ref, o_ref):
    h = h_ref[...]
    ps = pay_ref[...]
    cnt = ps[:, 4:5]
    inv = jnp.where(cnt > 0, 1.0 / jnp.maximum(cnt, 1.0), 0.0)
    agg = (agg_ref[0] + agg_ref[1]) * inv
    attr_mean = ps[:, :4] * inv
    add = jnp.where(
        cnt > 0,
        jnp.dot(attr_mean, we_ref[:4], preferred_element_type=jnp.float32)
        + be_ref[...],
        0.0)
    y = (jnp.dot(h, wt_ref[...], preferred_element_type=jnp.float32)
         + jnp.dot(agg, wb_ref[...], preferred_element_type=jnp.float32)
         + b_ref[...] + add)
    m = jnp.mean(y, axis=-1, keepdims=True)
    v = jnp.mean((y - m) * (y - m), axis=-1, keepdims=True)
    o_ref[...] = (y - m) * lax.rsqrt(v + 1e-5) * g_ref[...] + beta_ref[...]


def _tc_layer(h, agg, pay4, wt, wb, b, we8, be, gamma, beta):
    np_, d = h.shape
    blk = 2048
    grid = np_ // blk
    fixed = lambda i: (0, 0)
    out = pl.pallas_call(
        _tc_layer_body,
        grid=(grid,),
        in_specs=[
            pl.BlockSpec((blk, d), lambda i: (i, 0)),
            pl.BlockSpec((NC, blk, d), lambda i: (0, i, 0)),
            pl.BlockSpec((blk, NPAY), lambda i: (i, 0)),
            pl.BlockSpec((d, d), fixed),
            pl.BlockSpec((d, d), fixed),
            pl.BlockSpec((1, d), fixed),
            pl.BlockSpec((8, d), fixed),
            pl.BlockSpec((1, d), fixed),
            pl.BlockSpec((1, d), fixed),
            pl.BlockSpec((1, d), fixed),
        ],
        out_specs=pl.BlockSpec((blk, d), lambda i: (i, 0)),
        out_shape=jax.ShapeDtypeStruct((np_, d), jnp.float32),
    )(h, agg, pay4, wt, wb, b, we8, be, gamma, beta)
    return out


def kernel(x, edge_index, edge_attr, W1, b1, W2, b2, We, be, gamma, beta):
    n, d = x.shape
    e = edge_index.shape[1]
    de = edge_attr.shape[1]
    src = edge_index[0]
    dst = edge_index[1]
    np_ = ((n + NS * CHUNK - 1) // (NS * CHUNK)) * (NS * CHUNK)
    xp = jnp.concatenate([x, jnp.zeros((np_ - n, d), jnp.float32)], axis=0)
    we8 = jnp.concatenate(
        [We, jnp.zeros((8 - de, d), jnp.float32)], axis=0)

    pay = _sc_pay_pass(np_, dst, edge_attr)
    rows = (NPAY * np_) // 128
    pay4 = _tc_payred(pay.reshape(NC, NS, rows, 128)).reshape(np_, NPAY)
    agg1 = _sc_agg_pass(xp, src, dst)
    h1 = _tc_layer(xp, agg1, pay4, W1[:d], W1[d:], b1[None, :], we8,
                   be[None, :], gamma[None, :], beta[None, :])
    agg2 = _sc_agg_pass(h1, src, dst)
    h2 = _tc_layer(h1, agg2, pay4, W2[:d], W2[d:], b2[None, :], we8,
                   be[None, :], gamma[None, :], beta[None, :])
    return h2[:n]


# speedup vs baseline: 10.1253x; 1.1822x over previous
import functools

import jax
import jax.numpy as jnp
from jax import lax
from jax.experimental import pallas as pl
from jax.experimental.pallas import tpu as pltpu
from jax.experimental.pallas import tpu_sc as plsc

NC = 2
NS = 16
NW = NC * NS
CHUNK = 80
CHUNKP = 512
NPAY = 5


def _mesh():
    return plsc.VectorSubcoreMesh(core_axis_name="c", subcore_axis_name="s",
                                  num_cores=NC, num_subcores=NS)


NBUF = 2


def _sc_agg(np_, e, d, h_hbm, srcs3, dsts, out_agg, acc, sall, *bufs):
    didx = bufs[0:NBUF]
    rows = bufs[NBUF:2 * NBUF]
    sems = bufs[2 * NBUF:3 * NBUF]
    ssems = bufs[3 * NBUF:4 * NBUF]
    c = lax.axis_index("c")
    s = lax.axis_index("s")
    wid = s * NC + c
    epw = e // NW
    nch = epw // CHUNK
    rpt = np_ // NS
    nstripe = rpt // CHUNK

    def _zero(i, _):
        rows[0][i // 8, pl.ds((i % 8) * 16, 16)] = jnp.zeros((16,),
                                                             jnp.float32)
        return _
    lax.fori_loop(0, CHUNK * (d // 16), _zero, None)
    r0 = s * rpt
    for k in range(nstripe):
        pltpu.sync_copy(rows[0], acc.at[pl.ds(r0 + k * CHUNK, CHUNK), :])
    plsc.subcore_barrier()

    pltpu.sync_copy(srcs3.at[wid], sall)
    ebase = wid * epw
    for b in range(min(NBUF, nch)):
        pltpu.sync_copy(dsts.at[pl.ds(ebase + b * CHUNK, CHUNK)], didx[b])
        pltpu.async_copy(h_hbm.at[sall.at[b]], rows[b], sems[b])
    nq = nch // NBUF

    def _ring(q, _):
        for b in range(NBUF):
            ch = q * NBUF + b
            pltpu.make_async_copy(h_hbm.at[sall.at[ch]], rows[b],
                                  sems[b]).wait()
            pltpu.async_copy(rows[b], acc.at[didx[b]], ssems[b], add=True)
            nxt = ch + NBUF
            @pl.when(nxt < nch)
            def _(b=b, nxt=nxt):
                pltpu.make_async_copy(rows[b], acc.at[didx[b]],
                                      ssems[b]).wait()
                pltpu.sync_copy(dsts.at[pl.ds(ebase + nxt * CHUNK, CHUNK)],
                                didx[b])
                pltpu.async_copy(h_hbm.at[sall.at[nxt]], rows[b], sems[b])
        return _
    lax.fori_loop(0, nq, _ring, None)
    for ch in range(nq * NBUF, nch):
        b = ch % NBUF
        pltpu.make_async_copy(h_hbm.at[sall.at[ch]], rows[b], sems[b]).wait()
        pltpu.async_copy(rows[b], acc.at[didx[b]], ssems[b], add=True)
    for b in range(min(NBUF, nch)):
        pltpu.make_async_copy(rows[b], acc.at[didx[b]], ssems[b]).wait()
    plsc.subcore_barrier()

    for k in range(nstripe):
        rk = r0 + k * CHUNK
        pltpu.sync_copy(acc.at[pl.ds(rk, CHUNK), :], rows[0])
        pltpu.sync_copy(rows[0], out_agg.at[c, pl.ds(rk, CHUNK), :])


def _sc_agg_pass(h, srcs, dsts):
    np_, d = h.shape
    e = srcs.shape[0]
    epw = e // NW
    nch = epw // CHUNK
    scratch = [
        pltpu.VMEM_SHARED((np_, d), jnp.float32),
        pltpu.VMEM((nch, CHUNK), jnp.int32),
    ]
    scratch += [pltpu.VMEM((CHUNK,), jnp.int32) for _ in range(NBUF)]
    scratch += [pltpu.VMEM((CHUNK, d), jnp.float32) for _ in range(NBUF)]
    scratch += [pltpu.SemaphoreType.DMA for _ in range(2 * NBUF)]
    fn = pl.kernel(
        functools.partial(_sc_agg, np_, e, d),
        out_type=jax.ShapeDtypeStruct((NC, np_, d), jnp.float32),
        mesh=_mesh(),
        scratch_types=scratch,
        name="sc_agg")
    return fn(h, srcs.reshape(NW, nch, CHUNK), dsts)


def _sc_pay(np_, e, dsts, a0, a1, a2, a3, out_pay, acc5, didx, av0, av1, av2,
            av3, semp):
    c = lax.axis_index("c")
    s = lax.axis_index("s")
    wid = s * NC + c
    ncht = e // CHUNKP
    nch = ncht // NW + jnp.where(wid < ncht % NW, 1, 0)

    def _zero(i, _):
        acc5[pl.ds(i * 16, 16)] = jnp.zeros((16,), jnp.float32)
        return _
    lax.fori_loop(0, (NPAY * np_) // 16, _zero, None)

    ones = jnp.ones((16,), jnp.float32)

    def _chunk(t, _):
        e0 = (wid + NW * t) * CHUNKP
        cps = [pltpu.async_copy(dsts.at[pl.ds(e0, CHUNKP)], didx, semp),
               pltpu.async_copy(a0.at[pl.ds(e0, CHUNKP)], av0, semp),
               pltpu.async_copy(a1.at[pl.ds(e0, CHUNKP)], av1, semp),
               pltpu.async_copy(a2.at[pl.ds(e0, CHUNKP)], av2, semp),
               pltpu.async_copy(a3.at[pl.ds(e0, CHUNKP)], av3, semp)]
        for cp in cps:
            cp.wait()
        def _grp(g, __):
            sl = pl.ds(g * 16, 16)
            base = didx[sl] * NPAY
            plsc.addupdate_scatter(acc5, [base], av0[sl])
            plsc.addupdate_scatter(acc5, [base + 1], av1[sl])
            plsc.addupdate_scatter(acc5, [base + 2], av2[sl])
            plsc.addupdate_scatter(acc5, [base + 3], av3[sl])
            plsc.addupdate_scatter(acc5, [base + 4], ones)
            return __
        lax.fori_loop(0, CHUNKP // 16, _grp, None)
        return _
    lax.fori_loop(0, nch, _chunk, None)

    pltpu.sync_copy(acc5, out_pay.at[c, s])


def _sc_pay_pass(np_, dsts, edge_attr):
    e = dsts.shape[0]
    fn = pl.kernel(
        functools.partial(_sc_pay, np_, e),
        out_type=jax.ShapeDtypeStruct((NC, NS, NPAY * np_), jnp.float32),
        mesh=_mesh(),
        scratch_types=[
            pltpu.VMEM((NPAY * np_,), jnp.float32),
            pltpu.VMEM((CHUNKP,), jnp.int32),
            pltpu.VMEM((CHUNKP,), jnp.float32),
            pltpu.VMEM((CHUNKP,), jnp.float32),
            pltpu.VMEM((CHUNKP,), jnp.float32),
            pltpu.VMEM((CHUNKP,), jnp.float32),
            pltpu.SemaphoreType.DMA,
        ],
        compiler_params=pltpu.CompilerParams(needs_layout_passes=False),
        name="sc_pay")
    return fn(dsts, edge_attr[:, 0], edge_attr[:, 1], edge_attr[:, 2],
              edge_attr[:, 3])


def _tc_payred_body(p_ref, o_ref):
    acc = p_ref[0, 0]
    for i in range(NC):
        for j in range(NS):
            if i or j:
                acc = acc + p_ref[i, j]
    o_ref[...] = acc


def _tc_payred(pay):
    nc, ns, rows, d = pay.shape
    blk = 80
    return pl.pallas_call(
        _tc_payred_body,
        grid=(rows // blk,),
        in_specs=[pl.BlockSpec((NC, NS, blk, d), lambda i: (0, 0, i, 0))],
        out_specs=pl.BlockSpec((blk, d), lambda i: (i, 0)),
        out_shape=jax.ShapeDtypeStruct((rows, d), jnp.float32),
    )(pay)


def _tc_layer_body(h_ref, agg_ref, pay_ref, wt_ref, wb_ref, b_ref, we_ref,
                   be_ref, g_ref, beta_ref, o_ref):
    h = h_ref[...]
    ps = pay_ref[...]
    cnt = ps[:, 4:5]
    inv = jnp.where(cnt > 0, 1.0 / jnp.maximum(cnt, 1.0), 0.0)
    agg = (agg_ref[0] + agg_ref[1]) * inv
    attr_mean = ps[:, :4] * inv
    add = jnp.where(
        cnt > 0,
        jnp.dot(attr_mean, we_ref[:4], preferred_element_type=jnp.float32)
        + be_ref[...],
        0.0)
    y = (jnp.dot(h, wt_ref[...], preferred_element_type=jnp.float32)
         + jnp.dot(agg, wb_ref[...], preferred_element_type=jnp.float32)
         + b_ref[...] + add)
    m = jnp.mean(y, axis=-1, keepdims=True)
    v = jnp.mean((y - m) * (y - m), axis=-1, keepdims=True)
    o_ref[...] = (y - m) * lax.rsqrt(v + 1e-5) * g_ref[...] + beta_ref[...]


def _tc_layer(h, agg, pay4, wt, wb, b, we8, be, gamma, beta):
    np_, d = h.shape
    blk = 2048
    grid = np_ // blk
    fixed = lambda i: (0, 0)
    out = pl.pallas_call(
        _tc_layer_body,
        grid=(grid,),
        in_specs=[
            pl.BlockSpec((blk, d), lambda i: (i, 0)),
            pl.BlockSpec((NC, blk, d), lambda i: (0, i, 0)),
            pl.BlockSpec((blk, NPAY), lambda i: (i, 0)),
            pl.BlockSpec((d, d), fixed),
            pl.BlockSpec((d, d), fixed),
            pl.BlockSpec((1, d), fixed),
            pl.BlockSpec((8, d), fixed),
            pl.BlockSpec((1, d), fixed),
            pl.BlockSpec((1, d), fixed),
            pl.BlockSpec((1, d), fixed),
        ],
        out_specs=pl.BlockSpec((blk, d), lambda i: (i, 0)),
        out_shape=jax.ShapeDtypeStruct((np_, d), jnp.float32),
    )(h, agg, pay4, wt, wb, b, we8, be, gamma, beta)
    return out


def kernel(x, edge_index, edge_attr, W1, b1, W2, b2, We, be, gamma, beta):
    n, d = x.shape
    e = edge_index.shape[1]
    de = edge_attr.shape[1]
    src = edge_index[0]
    dst = edge_index[1]
    np_ = ((n + NS * CHUNK - 1) // (NS * CHUNK)) * (NS * CHUNK)
    xp = jnp.concatenate([x, jnp.zeros((np_ - n, d), jnp.float32)], axis=0)
    we8 = jnp.concatenate(
        [We, jnp.zeros((8 - de, d), jnp.float32)], axis=0)

    pay = _sc_pay_pass(np_, dst, edge_attr)
    rows = (NPAY * np_) // 128
    pay4 = _tc_payred(pay.reshape(NC, NS, rows, 128)).reshape(np_, NPAY)
    agg1 = _sc_agg_pass(xp, src, dst)
    h1 = _tc_layer(xp, agg1, pay4, W1[:d], W1[d:], b1[None, :], we8,
                   be[None, :], gamma[None, :], beta[None, :])
    agg2 = _sc_agg_pass(h1, src, dst)
    h2 = _tc_layer(h1, agg2, pay4, W2[:d], W2[d:], b2[None, :], we8,
                   be[None, :], gamma[None, :], beta[None, :])
    return h2[:n]


# speedup vs baseline: 10.6557x; 1.0524x over previous
import functools

import jax
import jax.numpy as jnp
from jax import lax
from jax.experimental import pallas as pl
from jax.experimental.pallas import tpu as pltpu
from jax.experimental.pallas import tpu_sc as plsc

NC = 2
NS = 16
NW = NC * NS
CHUNK = 80
CHUNKP = 512
NPAY = 5


def _mesh():
    return plsc.VectorSubcoreMesh(core_axis_name="c", subcore_axis_name="s",
                                  num_cores=NC, num_subcores=NS)


NBUF = 3


def _sc_agg(np_, e, d, h_hbm, srcs3, dsts, out_agg, acc, sall, *bufs):
    didx = bufs[0:NBUF]
    rows = bufs[NBUF:2 * NBUF]
    sems = bufs[2 * NBUF:3 * NBUF]
    ssems = bufs[3 * NBUF:4 * NBUF]
    c = lax.axis_index("c")
    s = lax.axis_index("s")
    wid = s * NC + c
    epw = e // NW
    nch = epw // CHUNK
    rpt = np_ // NS
    nstripe = rpt // CHUNK

    def _zero(i, _):
        rows[0][i // 8, pl.ds((i % 8) * 16, 16)] = jnp.zeros((16,),
                                                             jnp.float32)
        return _
    lax.fori_loop(0, CHUNK * (d // 16), _zero, None)
    r0 = s * rpt
    for k in range(nstripe):
        pltpu.sync_copy(rows[0], acc.at[pl.ds(r0 + k * CHUNK, CHUNK), :])
    plsc.subcore_barrier()

    pltpu.sync_copy(srcs3.at[wid], sall)
    ebase = wid * epw
    for b in range(min(NBUF, nch)):
        pltpu.sync_copy(dsts.at[pl.ds(ebase + b * CHUNK, CHUNK)], didx[b])
        pltpu.async_copy(h_hbm.at[sall.at[b]], rows[b], sems[b])
    nq = nch // NBUF

    def _ring(q, _):
        for b in range(NBUF):
            ch = q * NBUF + b
            pltpu.make_async_copy(h_hbm.at[sall.at[ch]], rows[b],
                                  sems[b]).wait()
            pltpu.async_copy(rows[b], acc.at[didx[b]], ssems[b], add=True)
            nxt = ch + NBUF
            @pl.when(nxt < nch)
            def _(b=b, nxt=nxt):
                pltpu.make_async_copy(rows[b], acc.at[didx[b]],
                                      ssems[b]).wait()
                pltpu.sync_copy(dsts.at[pl.ds(ebase + nxt * CHUNK, CHUNK)],
                                didx[b])
                pltpu.async_copy(h_hbm.at[sall.at[nxt]], rows[b], sems[b])
        return _
    lax.fori_loop(0, nq, _ring, None)
    for ch in range(nq * NBUF, nch):
        b = ch % NBUF
        pltpu.make_async_copy(h_hbm.at[sall.at[ch]], rows[b], sems[b]).wait()
        pltpu.async_copy(rows[b], acc.at[didx[b]], ssems[b], add=True)
    for b in range(min(NBUF, nch)):
        pltpu.make_async_copy(rows[b], acc.at[didx[b]], ssems[b]).wait()
    plsc.subcore_barrier()

    for k in range(nstripe):
        rk = r0 + k * CHUNK
        pltpu.sync_copy(acc.at[pl.ds(rk, CHUNK), :], rows[0])
        pltpu.sync_copy(rows[0], out_agg.at[c, pl.ds(rk, CHUNK), :])


def _sc_agg_pass(h, srcs, dsts):
    np_, d = h.shape
    e = srcs.shape[0]
    epw = e // NW
    nch = epw // CHUNK
    scratch = [
        pltpu.VMEM_SHARED((np_, d), jnp.float32),
        pltpu.VMEM((nch, CHUNK), jnp.int32),
    ]
    scratch += [pltpu.VMEM((CHUNK,), jnp.int32) for _ in range(NBUF)]
    scratch += [pltpu.VMEM((CHUNK, d), jnp.float32) for _ in range(NBUF)]
    scratch += [pltpu.SemaphoreType.DMA for _ in range(2 * NBUF)]
    fn = pl.kernel(
        functools.partial(_sc_agg, np_, e, d),
        out_type=jax.ShapeDtypeStruct((NC, np_, d), jnp.float32),
        mesh=_mesh(),
        scratch_types=scratch,
        name="sc_agg")
    return fn(h, srcs.reshape(NW, nch, CHUNK), dsts)


def _sc_pay(np_, e, dsts, a0, a1, a2, a3, out_pay, acc5, didx, av0, av1, av2,
            av3, semp):
    c = lax.axis_index("c")
    s = lax.axis_index("s")
    wid = s * NC + c
    ncht = e // CHUNKP
    nch = ncht // NW + jnp.where(wid < ncht % NW, 1, 0)

    def _zero(i, _):
        acc5[pl.ds(i * 16, 16)] = jnp.zeros((16,), jnp.float32)
        return _
    lax.fori_loop(0, (NPAY * np_) // 16, _zero, None)

    ones = jnp.ones((16,), jnp.float32)

    def _chunk(t, _):
        e0 = (wid + NW * t) * CHUNKP
        cps = [pltpu.async_copy(dsts.at[pl.ds(e0, CHUNKP)], didx, semp),
               pltpu.async_copy(a0.at[pl.ds(e0, CHUNKP)], av0, semp),
               pltpu.async_copy(a1.at[pl.ds(e0, CHUNKP)], av1, semp),
               pltpu.async_copy(a2.at[pl.ds(e0, CHUNKP)], av2, semp),
               pltpu.async_copy(a3.at[pl.ds(e0, CHUNKP)], av3, semp)]
        for cp in cps:
            cp.wait()
        def _grp(g, __):
            sl = pl.ds(g * 16, 16)
            base = didx[sl] * NPAY
            plsc.addupdate_scatter(acc5, [base], av0[sl])
            plsc.addupdate_scatter(acc5, [base + 1], av1[sl])
            plsc.addupdate_scatter(acc5, [base + 2], av2[sl])
            plsc.addupdate_scatter(acc5, [base + 3], av3[sl])
            plsc.addupdate_scatter(acc5, [base + 4], ones)
            return __
        lax.fori_loop(0, CHUNKP // 16, _grp, None)
        return _
    lax.fori_loop(0, nch, _chunk, None)

    pltpu.sync_copy(acc5, out_pay.at[c, s])


def _sc_pay_pass(np_, dsts, edge_attr):
    e = dsts.shape[0]
    fn = pl.kernel(
        functools.partial(_sc_pay, np_, e),
        out_type=jax.ShapeDtypeStruct((NC, NS, NPAY * np_), jnp.float32),
        mesh=_mesh(),
        scratch_types=[
            pltpu.VMEM((NPAY * np_,), jnp.float32),
            pltpu.VMEM((CHUNKP,), jnp.int32),
            pltpu.VMEM((CHUNKP,), jnp.float32),
            pltpu.VMEM((CHUNKP,), jnp.float32),
            pltpu.VMEM((CHUNKP,), jnp.float32),
            pltpu.VMEM((CHUNKP,), jnp.float32),
            pltpu.SemaphoreType.DMA,
        ],
        compiler_params=pltpu.CompilerParams(needs_layout_passes=False),
        name="sc_pay")
    return fn(dsts, edge_attr[:, 0], edge_attr[:, 1], edge_attr[:, 2],
              edge_attr[:, 3])


def _tc_payred_body(p_ref, o_ref):
    acc = p_ref[0, 0]
    for i in range(NC):
        for j in range(NS):
            if i or j:
                acc = acc + p_ref[i, j]
    o_ref[...] = acc


def _tc_payred(pay):
    nc, ns, rows, d = pay.shape
    blk = 80
    return pl.pallas_call(
        _tc_payred_body,
        grid=(rows // blk,),
        in_specs=[pl.BlockSpec((NC, NS, blk, d), lambda i: (0, 0, i, 0))],
        out_specs=pl.BlockSpec((blk, d), lambda i: (i, 0)),
        out_shape=jax.ShapeDtypeStruct((rows, d), jnp.float32),
    )(pay)


def _tc_layer_body(h_ref, agg_ref, pay_ref, wt_ref, wb_ref, b_ref, we_ref,
                   be_ref, g_ref, beta_ref, o_ref):
    h = h_ref[...]
    ps = pay_ref[...]
    cnt = ps[:, 4:5]
    inv = jnp.where(cnt > 0, 1.0 / jnp.maximum(cnt, 1.0), 0.0)
    agg = (agg_ref[0] + agg_ref[1]) * inv
    attr_mean = ps[:, :4] * inv
    add = jnp.where(
        cnt > 0,
        jnp.dot(attr_mean, we_ref[:4], preferred_element_type=jnp.float32)
        + be_ref[...],
        0.0)
    y = (jnp.dot(h, wt_ref[...], preferred_element_type=jnp.float32)
         + jnp.dot(agg, wb_ref[...], preferred_element_type=jnp.float32)
         + b_ref[...] + add)
    m = jnp.mean(y, axis=-1, keepdims=True)
    v = jnp.mean((y - m) * (y - m), axis=-1, keepdims=True)
    o_ref[...] = (y - m) * lax.rsqrt(v + 1e-5) * g_ref[...] + beta_ref[...]


def _tc_layer(h, agg, pay4, wt, wb, b, we8, be, gamma, beta):
    np_, d = h.shape
    blk = 2048
    grid = np_ // blk
    fixed = lambda i: (0, 0)
    out = pl.pallas_call(
        _tc_layer_body,
        grid=(grid,),
        in_specs=[
            pl.BlockSpec((blk, d), lambda i: (i, 0)),
            pl.BlockSpec((NC, blk, d), lambda i: (0, i, 0)),
            pl.BlockSpec((blk, NPAY), lambda i: (i, 0)),
            pl.BlockSpec((d, d), fixed),
            pl.BlockSpec((d, d), fixed),
            pl.BlockSpec((1, d), fixed),
            pl.BlockSpec((8, d), fixed),
            pl.BlockSpec((1, d), fixed),
            pl.BlockSpec((1, d), fixed),
            pl.BlockSpec((1, d), fixed),
        ],
        out_specs=pl.BlockSpec((blk, d), lambda i: (i, 0)),
        out_shape=jax.ShapeDtypeStruct((np_, d), jnp.float32),
    )(h, agg, pay4, wt, wb, b, we8, be, gamma, beta)
    return out


def kernel(x, edge_index, edge_attr, W1, b1, W2, b2, We, be, gamma, beta):
    n, d = x.shape
    e = edge_index.shape[1]
    de = edge_attr.shape[1]
    src = edge_index[0]
    dst = edge_index[1]
    np_ = ((n + NS * CHUNK - 1) // (NS * CHUNK)) * (NS * CHUNK)
    xp = jnp.concatenate([x, jnp.zeros((np_ - n, d), jnp.float32)], axis=0)
    we8 = jnp.concatenate(
        [We, jnp.zeros((8 - de, d), jnp.float32)], axis=0)

    pay = _sc_pay_pass(np_, dst, edge_attr)
    rows = (NPAY * np_) // 128
    pay4 = _tc_payred(pay.reshape(NC, NS, rows, 128)).reshape(np_, NPAY)
    agg1 = _sc_agg_pass(xp, src, dst)
    h1 = _tc_layer(xp, agg1, pay4, W1[:d], W1[d:], b1[None, :], we8,
                   be[None, :], gamma[None, :], beta[None, :])
    agg2 = _sc_agg_pass(h1, src, dst)
    h2 = _tc_layer(h1, agg2, pay4, W2[:d], W2[d:], b2[None, :], we8,
                   be[None, :], gamma[None, :], beta[None, :])
    return h2[:n]
